# baseline (speedup 1.0000x reference)
import functools

import jax
import jax.numpy as jnp
from jax import lax
from jax.experimental import pallas as pl
from jax.experimental.pallas import tpu as pltpu
from jax.experimental.pallas import tpu_sc as plsc

N_NODES = 10000
N_EDGES = 320000
HID = 128

_NC = 2
_NS = 16
_NW = _NC * _NS
_EPW = N_EDGES // _NW
_CH = 128
_NFULL = _EPW // _CH
_REM = _EPW - _NFULL * _CH
_NBUF = 3
_SNBUF = 2
_RPT = 1000
_NDRAIN = N_NODES // _RPT

_EH = N_EDGES // 2
_EB = 1280
_EGRID = N_EDGES // _EB
_EGRID_H = _EH // _EB
_NB = 2000
_NGRID = N_NODES // _NB


def _softplus(x):
  return jnp.maximum(x, 0.0) + jnp.log1p(jnp.exp(-jnp.abs(x)))


def _gather_body(n_str, epw, base0, ch, nbuf, table, *args):
  nfull = epw // ch
  rem = epw - nfull * ch
  idxs = args[:n_str]
  outs = args[n_str:2 * n_str]
  sc = args[2 * n_str:]
  stab = sc[0]
  sc = sc[1:]
  idxbuf = sc[:nbuf * n_str]
  rowbuf = sc[nbuf * n_str:2 * nbuf * n_str]
  sems = sc[2 * nbuf * n_str:]
  sem_i = sems[:nbuf * n_str]
  sem_g = sems[nbuf * n_str:2 * nbuf * n_str]
  sem_o = sems[2 * nbuf * n_str:3 * nbuf * n_str]

  s = lax.axis_index("s")
  wid = s * _NC + lax.axis_index("c")
  ibase = base0 + wid * epw
  obase = wid * epw

  @pl.when(s < _NDRAIN)
  def _():
    pltpu.sync_copy(table.at[pl.ds(s * _RPT, _RPT)],
                    stab.at[pl.ds(s * _RPT, _RPT)])
  plsc.subcore_barrier()

  def sl(n, b):
    return n * nbuf + b

  def start_idx(w, n, b):
    return pltpu.async_copy(
        idxs[n].at[pl.ds(ibase + w * ch, ch)], idxbuf[sl(n, b)],
        sem_i[sl(n, b)])

  def start_gather(n, b):
    return pltpu.async_copy(
        stab.at[idxbuf[sl(n, b)]], rowbuf[sl(n, b)], sem_g[sl(n, b)])

  def start_out(w, n, b):
    return pltpu.async_copy(
        rowbuf[sl(n, b)], outs[n].at[pl.ds(obase + w * ch, ch)],
        sem_o[sl(n, b)])

  hi, hg, ho = {}, {}, {}
  for w in range(min(nbuf, nfull)):
    for n in range(n_str):
      hi[(w, n)] = start_idx(w, n, w % nbuf)
  for w in range(nfull):
    b = w % nbuf
    for n in range(n_str):
      if w >= nbuf:
        ho[(w - nbuf, n)].wait()
      hi[(w, n)].wait()
      hg[(w, n)] = start_gather(n, b)
    for n in range(n_str):
      if w >= 1:
        hg[(w - 1, n)].wait()
        ho[(w - 1, n)] = start_out(w - 1, n, (w - 1) % nbuf)
        if w + nbuf - 1 < nfull:
          hi[(w + nbuf - 1, n)] = start_idx(w + nbuf - 1, n, (w - 1) % nbuf)
  for n in range(n_str):
    hg[(nfull - 1, n)].wait()
    ho[(nfull - 1, n)] = start_out(nfull - 1, n, (nfull - 1) % nbuf)
  for w in range(max(0, nfull - nbuf), nfull):
    for n in range(n_str):
      ho[(w, n)].wait()
  if rem:
    for n in range(n_str):
      pltpu.sync_copy(idxs[n].at[pl.ds(ibase + nfull * ch, rem)],
                      idxbuf[sl(n, 0)].at[pl.ds(0, rem)])
      pltpu.async_copy(stab.at[idxbuf[sl(n, 0)].at[pl.ds(0, rem)]],
                       rowbuf[sl(n, 0)].at[pl.ds(0, rem)],
                       sem_g[sl(n, 0)]).wait()
      pltpu.sync_copy(rowbuf[sl(n, 0)].at[pl.ds(0, rem)],
                      outs[n].at[pl.ds(obase + nfull * ch, rem)])


def _make_gather(n_str, n_edges=N_EDGES, base0=0, ch=_CH, nbuf=_NBUF):
  mesh = plsc.VectorSubcoreMesh(core_axis_name="c", subcore_axis_name="s")
  epw = n_edges // _NW
  scratch = (
      [pltpu.VMEM_SHARED((N_NODES, HID), jnp.float32)]
      + [pltpu.VMEM((ch,), jnp.int32)] * (nbuf * n_str)
      + [pltpu.VMEM((ch, HID), jnp.float32)] * (nbuf * n_str)
      + [pltpu.SemaphoreType.DMA] * (3 * nbuf * n_str)
  )
  out = (jax.ShapeDtypeStruct((n_edges, HID), jnp.float32),) * n_str
  return pl.kernel(
      functools.partial(_gather_body, n_str, epw, base0, ch, nbuf),
      out_type=out,
      mesh=mesh,
      scratch_types=scratch,
  )


def _scatter_body(epw, base0, msg, dst, zin, out, agg,
                  *sc):
  nfull = epw // _CH
  rem = epw - nfull * _CH
  dstbuf = sc[:_SNBUF]
  updbuf = sc[_SNBUF:2 * _SNBUF]
  remidx = sc[2 * _SNBUF]
  remupd = sc[2 * _SNBUF + 1]
  rest = sc[2 * _SNBUF + 2:]
  sem_i = rest[:_SNBUF]
  sem_u = rest[_SNBUF:2 * _SNBUF]
  sem_s = rest[2 * _SNBUF:3 * _SNBUF]

  c = lax.axis_index("c")
  s = lax.axis_index("s")
  wid = s * _NC + c
  ibase = base0 + wid * epw
  mbase = wid * epw

  @pl.when(s < _NDRAIN)
  def _():
    pltpu.sync_copy(zin, agg.at[pl.ds(s * _RPT, _RPT)])
  plsc.subcore_barrier()

  def start_idx(w, b):
    return pltpu.async_copy(dst.at[pl.ds(ibase + w * _CH, _CH)], dstbuf[b],
                            sem_i[b])

  def start_upd(w, b):
    return pltpu.async_copy(msg.at[pl.ds(mbase + w * _CH, _CH)], updbuf[b],
                            sem_u[b])

  def start_scatter(b):
    return pltpu.async_copy(updbuf[b], agg.at[dstbuf[b]], sem_s[b], add=True)

  hi, hu, hs = {}, {}, {}
  for w in range(min(_SNBUF, nfull)):
    hi[w] = start_idx(w, w % _SNBUF)
    hu[w] = start_upd(w, w % _SNBUF)
  for w in range(nfull):
    b = w % _SNBUF
    hi[w].wait()
    hu[w].wait()
    hs[w] = start_scatter(b)
    if w >= 1:
      hs[w - 1].wait()
      if w + _SNBUF - 1 < nfull:
        hi[w + _SNBUF - 1] = start_idx(w + _SNBUF - 1, (w - 1) % _SNBUF)
        hu[w + _SNBUF - 1] = start_upd(w + _SNBUF - 1, (w - 1) % _SNBUF)
  hs[nfull - 1].wait()
  if rem:
    pltpu.sync_copy(dst.at[pl.ds(ibase + nfull * _CH, rem)], remidx)
    pltpu.sync_copy(msg.at[pl.ds(mbase + nfull * _CH, rem)], remupd)
    pltpu.async_copy(remupd, agg.at[remidx], sem_s[0], add=True).wait()
  plsc.subcore_barrier()
  @pl.when(s < _NDRAIN)
  def _():
    pltpu.sync_copy(agg.at[pl.ds(s * _RPT, _RPT)],
                    out.at[pl.ds(c * N_NODES + s * _RPT, _RPT)])


def _make_scatter(n_edges=N_EDGES, base0=0):
  mesh = plsc.VectorSubcoreMesh(core_axis_name="c", subcore_axis_name="s")
  epw = n_edges // _NW
  scratch = (
      [pltpu.VMEM_SHARED((N_NODES, HID), jnp.float32)]
      + [pltpu.VMEM((_CH,), jnp.int32)] * _SNBUF
      + [pltpu.VMEM((_CH, HID), jnp.float32)] * _SNBUF
      + [pltpu.VMEM((max(epw % _CH, 8),), jnp.int32),
         pltpu.VMEM((max(epw % _CH, 8), HID), jnp.float32)]
      + [pltpu.SemaphoreType.DMA] * (3 * _SNBUF)
  )
  return pl.kernel(
      functools.partial(_scatter_body, epw, base0),
      out_type=jax.ShapeDtypeStruct((2 * N_NODES, HID), jnp.float32),
      mesh=mesh,
      scratch_types=scratch,
  )


def _ea_block(xr, eattr, a1, d1, w2, g2, d2):
  xp = lax.dot_general(xr, a1, (((0,), (0,)), ((), ())),
                       preferred_element_type=jnp.float32)
  d = _softplus(xp + d1)
  e2 = jnp.dot(d, w2, preferred_element_type=jnp.float32)
  return (e2 * g2 + d2) * eattr


def _msg_body(hs_ref, xr_ref, eattr_ref, a1_ref, d1_ref, w2_ref, g2_ref,
              d2_ref, out_ref):
  ea = _ea_block(xr_ref[...], eattr_ref[...], a1_ref[...], d1_ref[...],
                 w2_ref[...], g2_ref[...], d2_ref[...])
  out_ref[...] = _softplus(hs_ref[...] + ea)


def _node_body(act, a0_ref, a1_ref, b0_ref, b1_ref, h_ref, w_ref, g_ref,
               d_ref, out_ref):
  tot = ((a0_ref[...] + a1_ref[...]) + (b0_ref[...] + b1_ref[...])
         + h_ref[...])
  y = jnp.dot(tot, w_ref[...], preferred_element_type=jnp.float32)
  y = y * g_ref[...] + d_ref[...]
  out_ref[...] = _softplus(y) if act else y


def _pair_body(hr_ref, hc_ref, xr_ref, eattr_ref, a1_ref, d1_ref, wd2_ref,
               gd2_ref, dd2_ref, w1a_ref, w1b_ref, g1_ref, e1_ref,
               w2_ref, g2_ref, e2_ref, w3_ref, g3_ref, e3_ref, out_ref):
  ea = _ea_block(xr_ref[...], eattr_ref[...], a1_ref[...], d1_ref[...],
                 wd2_ref[...], gd2_ref[...], dd2_ref[...])
  prod = hr_ref[...] * hc_ref[...]
  y = (jnp.dot(prod, w1a_ref[...], preferred_element_type=jnp.float32)
       + jnp.dot(ea, w1b_ref[...], preferred_element_type=jnp.float32))
  p1 = _softplus(y * g1_ref[...] + e1_ref[...])
  y2 = jnp.dot(p1, w2_ref[...], preferred_element_type=jnp.float32)
  p2 = _softplus(y2 * g2_ref[...] + e2_ref[...])
  y3 = jnp.dot(p2, w3_ref[...], preferred_element_type=jnp.float32)
  out_ref[...] = y3 * g3_ref[...] + e3_ref[...]


def _vec_spec(d):
  return pl.BlockSpec((1, d), lambda i: (0, 0))


def _mat_spec(r, c):
  return pl.BlockSpec((r, c), lambda i: (0, 0))


def _edge_spec(d):
  return pl.BlockSpec((_EB, d), lambda i: (i, 0))


def _gate_bias(p, t):
  tt = t.reshape(1, 1)
  g = jax.nn.sigmoid(tt @ p["Wg"] + p["bg"])
  b = tt @ p["Wb"]
  return g, b


def kernel(t, x, node_attr, edge_attr, edge_index, params):
  f32 = jnp.float32
  src = edge_index[0]
  dst = edge_index[1]

  g1, c1 = _gate_bias(params["d_fc1"], t)
  g2, c2 = _gate_bias(params["d_fc2"], t)
  p1, p2 = params["d_fc1"], params["d_fc2"]
  a1 = p1["W"][0:1] * g1
  d1 = p1["b"][None] * g1 + c1
  w2 = p2["W"]
  d2 = p2["b"][None] * g2 + c2

  gathers1 = [_make_gather(1, _EH, 0), _make_gather(1, _EH, _EH)]
  gathers2 = [_make_gather(2, _EH, 0, ch=96, nbuf=2),
              _make_gather(2, _EH, _EH, ch=96, nbuf=2)]
  scatters = [_make_scatter(_EH, 0), _make_scatter(_EH, _EH)]
  zin = jnp.zeros((_RPT, HID), f32)

  xr = x.reshape(1, N_EDGES)

  def _msg_half(hs, half):
    off = half * _EGRID_H
    return pl.pallas_call(
        _msg_body,
        grid=(_EGRID_H,),
        in_specs=[
            pl.BlockSpec((_EB, HID), lambda i: (i, 0)),
            pl.BlockSpec((1, _EB), lambda i, o=off: (0, i + o)),
            pl.BlockSpec((_EB, HID), lambda i, o=off: (i + o, 0)),
            _vec_spec(HID), _vec_spec(HID),
            _mat_spec(HID, HID),
            _vec_spec(HID), _vec_spec(HID),
        ],
        out_specs=pl.BlockSpec((_EB, HID), lambda i: (i, 0)),
        out_shape=jax.ShapeDtypeStruct((_EH, HID), f32),
    )(hs, xr, edge_attr, a1, d1, w2, g2, d2)

  h = node_attr
  for li, name in enumerate(("conv1", "conv2", "conv3")):
    pc = params[name]
    gk, ck = _gate_bias(pc, t)
    dk = pc["b"][None] * gk + ck
    (hs0,) = gathers1[0](h, src)
    (hs1,) = gathers1[1](h, src)
    msg0 = _msg_half(hs0, 0)
    msg1 = _msg_half(hs1, 1)
    parts0 = scatters[0](msg0, dst, zin)
    parts1 = scatters[1](msg1, dst, zin)
    h = pl.pallas_call(
        functools.partial(_node_body, li < 2),
        grid=(_NGRID,),
        in_specs=[
            pl.BlockSpec((_NB, HID), lambda i: (i, 0)),
            pl.BlockSpec((_NB, HID), lambda i: (i + _NGRID, 0)),
            pl.BlockSpec((_NB, HID), lambda i: (i, 0)),
            pl.BlockSpec((_NB, HID), lambda i: (i + _NGRID, 0)),
            pl.BlockSpec((_NB, HID), lambda i: (i, 0)),
            _mat_spec(HID, HID),
            _vec_spec(HID), _vec_spec(HID),
        ],
        out_specs=pl.BlockSpec((_NB, HID), lambda i: (i, 0)),
        out_shape=jax.ShapeDtypeStruct((N_NODES, HID), f32),
    )(parts0, parts0, parts1, parts1, h, pc["W"], gk, dk)


  pf1, pf2, pf3 = params["out_fc1"], params["out_fc2"], params["out_fc3"]
  gf1, cf1 = _gate_bias(pf1, t)
  gf2, cf2 = _gate_bias(pf2, t)
  gf3, cf3 = _gate_bias(pf3, t)
  e1 = pf1["b"][None] * gf1 + cf1
  e2 = pf2["b"][None] * gf2 + cf2
  e3 = pf3["b"][None] * gf3 + cf3

  outs = []
  for half in range(2):
    hr, hc = gathers2[half](h, src, dst)
    off = half * _EGRID_H
    outs.append(pl.pallas_call(
        _pair_body,
        grid=(_EGRID_H,),
        in_specs=[
            pl.BlockSpec((_EB, HID), lambda i: (i, 0)),
            pl.BlockSpec((_EB, HID), lambda i: (i, 0)),
            pl.BlockSpec((1, _EB), lambda i, o=off: (0, i + o)),
            pl.BlockSpec((_EB, HID), lambda i, o=off: (i + o, 0)),
            _vec_spec(HID), _vec_spec(HID),
            _mat_spec(HID, HID),
            _vec_spec(HID), _vec_spec(HID),
            _mat_spec(HID, HID), _mat_spec(HID, HID),
            _vec_spec(HID), _vec_spec(HID),
            _mat_spec(HID, HID // 2),
            _vec_spec(HID // 2), _vec_spec(HID // 2),
            _mat_spec(HID // 2, 1),
            _vec_spec(1), _vec_spec(1),
        ],
        out_specs=pl.BlockSpec((_EB, 1), lambda i: (i, 0)),
        out_shape=jax.ShapeDtypeStruct((_EH, 1), f32),
    )(hr, hc, xr, edge_attr, a1, d1, w2, g2, d2,
      pf1["W"][:HID], pf1["W"][HID:], gf1, e1,
      pf2["W"], gf2, e2, pf3["W"], gf3, e3))
  return jnp.concatenate(outs, axis=0)


# speedup vs baseline: 1.1409x; 1.1409x over previous
import functools

import jax
import jax.numpy as jnp
from jax import lax
from jax.experimental import pallas as pl
from jax.experimental.pallas import tpu as pltpu
from jax.experimental.pallas import tpu_sc as plsc

N_NODES = 10000
N_EDGES = 320000
HID = 128

_NC = 2
_NS = 16
_NW = _NC * _NS
_EPW = N_EDGES // _NW
_CH = 128
_NFULL = _EPW // _CH
_REM = _EPW - _NFULL * _CH
_NBUF = 3
_SNBUF = 2
_RPT = 1000
_NDRAIN = N_NODES // _RPT

_EH = N_EDGES // 2
_EB = 2000
_EBX = 2560
_EGRIDX = N_EDGES // _EBX
_EGRID = N_EDGES // _EB
_EGRID_H = _EH // _EB
_NB = 2000
_NGRID = N_NODES // _NB


def _softplus(x):
  return jnp.maximum(x, 0.0) + jnp.log1p(jnp.exp(-jnp.abs(x)))


def _gather_body(n_str, epw, base0, ch, nbuf, table, *args):
  nfull = epw // ch
  rem = epw - nfull * ch
  idxs = args[:n_str]
  outs = args[n_str:2 * n_str]
  sc = args[2 * n_str:]
  stab = sc[0]
  sc = sc[1:]
  idxbuf = sc[:nbuf * n_str]
  rowbuf = sc[nbuf * n_str:2 * nbuf * n_str]
  sems = sc[2 * nbuf * n_str:]
  sem_i = sems[:nbuf * n_str]
  sem_g = sems[nbuf * n_str:2 * nbuf * n_str]
  sem_o = sems[2 * nbuf * n_str:3 * nbuf * n_str]

  s = lax.axis_index("s")
  wid = s * _NC + lax.axis_index("c")
  ibase = base0 + wid * epw
  obase = wid * epw

  @pl.when(s < _NDRAIN)
  def _():
    pltpu.sync_copy(table.at[pl.ds(s * _RPT, _RPT)],
                    stab.at[pl.ds(s * _RPT, _RPT)])
  plsc.subcore_barrier()

  def sl(n, b):
    return n * nbuf + b

  def start_idx(w, n, b):
    return pltpu.async_copy(
        idxs[n].at[pl.ds(ibase + w * ch, ch)], idxbuf[sl(n, b)],
        sem_i[sl(n, b)])

  def start_gather(n, b):
    return pltpu.async_copy(
        stab.at[idxbuf[sl(n, b)]], rowbuf[sl(n, b)], sem_g[sl(n, b)])

  def start_out(w, n, b):
    return pltpu.async_copy(
        rowbuf[sl(n, b)], outs[n].at[pl.ds(obase + w * ch, ch)],
        sem_o[sl(n, b)])

  hi, hg, ho = {}, {}, {}
  for w in range(min(nbuf, nfull)):
    for n in range(n_str):
      hi[(w, n)] = start_idx(w, n, w % nbuf)
  for w in range(nfull):
    b = w % nbuf
    for n in range(n_str):
      if w >= nbuf:
        ho[(w - nbuf, n)].wait()
      hi[(w, n)].wait()
      hg[(w, n)] = start_gather(n, b)
    for n in range(n_str):
      if w >= 1:
        hg[(w - 1, n)].wait()
        ho[(w - 1, n)] = start_out(w - 1, n, (w - 1) % nbuf)
        if w + nbuf - 1 < nfull:
          hi[(w + nbuf - 1, n)] = start_idx(w + nbuf - 1, n, (w - 1) % nbuf)
  for n in range(n_str):
    hg[(nfull - 1, n)].wait()
    ho[(nfull - 1, n)] = start_out(nfull - 1, n, (nfull - 1) % nbuf)
  for w in range(max(0, nfull - nbuf), nfull):
    for n in range(n_str):
      ho[(w, n)].wait()
  if rem:
    for n in range(n_str):
      pltpu.sync_copy(idxs[n].at[pl.ds(ibase + nfull * ch, rem)],
                      idxbuf[sl(n, 0)].at[pl.ds(0, rem)])
      pltpu.async_copy(stab.at[idxbuf[sl(n, 0)].at[pl.ds(0, rem)]],
                       rowbuf[sl(n, 0)].at[pl.ds(0, rem)],
                       sem_g[sl(n, 0)]).wait()
      pltpu.sync_copy(rowbuf[sl(n, 0)].at[pl.ds(0, rem)],
                      outs[n].at[pl.ds(obase + nfull * ch, rem)])


def _make_gather(n_str, n_edges=N_EDGES, base0=0, ch=_CH, nbuf=_NBUF):
  mesh = plsc.VectorSubcoreMesh(core_axis_name="c", subcore_axis_name="s")
  epw = n_edges // _NW
  scratch = (
      [pltpu.VMEM_SHARED((N_NODES, HID), jnp.float32)]
      + [pltpu.VMEM((ch,), jnp.int32)] * (nbuf * n_str)
      + [pltpu.VMEM((ch, HID), jnp.float32)] * (nbuf * n_str)
      + [pltpu.SemaphoreType.DMA] * (3 * nbuf * n_str)
  )
  out = (jax.ShapeDtypeStruct((n_edges, HID), jnp.float32),) * n_str
  return pl.kernel(
      functools.partial(_gather_body, n_str, epw, base0, ch, nbuf),
      out_type=out,
      mesh=mesh,
      scratch_types=scratch,
  )


def _scatter_body(epw, base0, msg, dst, zin, out, agg,
                  *sc):
  nfull = epw // _CH
  rem = epw - nfull * _CH
  dstbuf = sc[:_SNBUF]
  updbuf = sc[_SNBUF:2 * _SNBUF]
  remidx = sc[2 * _SNBUF]
  remupd = sc[2 * _SNBUF + 1]
  rest = sc[2 * _SNBUF + 2:]
  sem_i = rest[:_SNBUF]
  sem_u = rest[_SNBUF:2 * _SNBUF]
  sem_s = rest[2 * _SNBUF:3 * _SNBUF]

  c = lax.axis_index("c")
  s = lax.axis_index("s")
  wid = s * _NC + c
  ibase = base0 + wid * epw
  mbase = wid * epw

  @pl.when(s < _NDRAIN)
  def _():
    pltpu.sync_copy(zin, agg.at[pl.ds(s * _RPT, _RPT)])
  plsc.subcore_barrier()

  def start_idx(w, b):
    return pltpu.async_copy(dst.at[pl.ds(ibase + w * _CH, _CH)], dstbuf[b],
                            sem_i[b])

  def start_upd(w, b):
    return pltpu.async_copy(msg.at[pl.ds(mbase + w * _CH, _CH)], updbuf[b],
                            sem_u[b])

  def start_scatter(b):
    return pltpu.async_copy(updbuf[b], agg.at[dstbuf[b]], sem_s[b], add=True)

  hi, hu, hs = {}, {}, {}
  for w in range(min(_SNBUF, nfull)):
    hi[w] = start_idx(w, w % _SNBUF)
    hu[w] = start_upd(w, w % _SNBUF)
  for w in range(nfull):
    b = w % _SNBUF
    hi[w].wait()
    hu[w].wait()
    hs[w] = start_scatter(b)
    if w >= 1:
      hs[w - 1].wait()
      if w + _SNBUF - 1 < nfull:
        hi[w + _SNBUF - 1] = start_idx(w + _SNBUF - 1, (w - 1) % _SNBUF)
        hu[w + _SNBUF - 1] = start_upd(w + _SNBUF - 1, (w - 1) % _SNBUF)
  hs[nfull - 1].wait()
  if rem:
    pltpu.sync_copy(dst.at[pl.ds(ibase + nfull * _CH, rem)], remidx)
    pltpu.sync_copy(msg.at[pl.ds(mbase + nfull * _CH, rem)], remupd)
    pltpu.async_copy(remupd, agg.at[remidx], sem_s[0], add=True).wait()
  plsc.subcore_barrier()
  @pl.when(s < _NDRAIN)
  def _():
    pltpu.sync_copy(agg.at[pl.ds(s * _RPT, _RPT)],
                    out.at[pl.ds(c * N_NODES + s * _RPT, _RPT)])


def _make_scatter(n_edges=N_EDGES, base0=0):
  mesh = plsc.VectorSubcoreMesh(core_axis_name="c", subcore_axis_name="s")
  epw = n_edges // _NW
  scratch = (
      [pltpu.VMEM_SHARED((N_NODES, HID), jnp.float32)]
      + [pltpu.VMEM((_CH,), jnp.int32)] * _SNBUF
      + [pltpu.VMEM((_CH, HID), jnp.float32)] * _SNBUF
      + [pltpu.VMEM((max(epw % _CH, 8),), jnp.int32),
         pltpu.VMEM((max(epw % _CH, 8), HID), jnp.float32)]
      + [pltpu.SemaphoreType.DMA] * (3 * _SNBUF)
  )
  return pl.kernel(
      functools.partial(_scatter_body, epw, base0),
      out_type=jax.ShapeDtypeStruct((2 * N_NODES, HID), jnp.float32),
      mesh=mesh,
      scratch_types=scratch,
  )


def _ea_body(xr_ref, eattr_ref, a1_ref, d1_ref, w2_ref, g2_ref, d2_ref,
             out_ref):
  xcol = jnp.transpose(xr_ref[...])
  d = _softplus(xcol * a1_ref[...] + d1_ref[...])
  e2 = jnp.dot(d, w2_ref[...], preferred_element_type=jnp.float32)
  out_ref[...] = (e2 * g2_ref[...] + d2_ref[...]) * eattr_ref[...]


def _msg_body(hs_ref, ea_ref, out_ref):
  out_ref[...] = _softplus(hs_ref[...] + ea_ref[...])


def _node_body(act, a0_ref, a1_ref, b0_ref, b1_ref, h_ref, w_ref, g_ref,
               d_ref, out_ref):
  tot = ((a0_ref[...] + a1_ref[...]) + (b0_ref[...] + b1_ref[...])
         + h_ref[...])
  y = jnp.dot(tot, w_ref[...], preferred_element_type=jnp.float32)
  y = y * g_ref[...] + d_ref[...]
  out_ref[...] = _softplus(y) if act else y


def _pair_body(hr_ref, hc_ref, ea_ref, w1a_ref, w1b_ref, g1_ref, e1_ref,
               w2_ref, g2_ref, e2_ref, w3_ref, g3_ref, e3_ref, out_ref):
  prod = hr_ref[...] * hc_ref[...]
  y = (jnp.dot(prod, w1a_ref[...], preferred_element_type=jnp.float32)
       + jnp.dot(ea_ref[...], w1b_ref[...], preferred_element_type=jnp.float32))
  p1 = _softplus(y * g1_ref[...] + e1_ref[...])
  y2 = jnp.dot(p1, w2_ref[...], preferred_element_type=jnp.float32)
  p2 = _softplus(y2 * g2_ref[...] + e2_ref[...])
  y3 = jnp.dot(p2, w3_ref[...], preferred_element_type=jnp.float32)
  out_ref[...] = y3 * g3_ref[...] + e3_ref[...]


def _vec_spec(d):
  return pl.BlockSpec((1, d), lambda i: (0, 0))


def _mat_spec(r, c):
  return pl.BlockSpec((r, c), lambda i: (0, 0))


def _edge_spec(d):
  return pl.BlockSpec((_EB, d), lambda i: (i, 0))


def _gate_bias(p, t):
  tt = t.reshape(1, 1)
  g = jax.nn.sigmoid(tt @ p["Wg"] + p["bg"])
  b = tt @ p["Wb"]
  return g, b


def kernel(t, x, node_attr, edge_attr, edge_index, params):
  f32 = jnp.float32
  src = edge_index[0]
  dst = edge_index[1]

  g1, c1 = _gate_bias(params["d_fc1"], t)
  g2, c2 = _gate_bias(params["d_fc2"], t)
  p1, p2 = params["d_fc1"], params["d_fc2"]
  a1 = p1["W"][0:1] * g1
  d1 = p1["b"][None] * g1 + c1
  w2 = p2["W"]
  d2 = p2["b"][None] * g2 + c2

  gathers1 = [_make_gather(1, _EH, 0), _make_gather(1, _EH, _EH)]
  gathers2 = [_make_gather(2, _EH, 0, ch=96, nbuf=2),
              _make_gather(2, _EH, _EH, ch=96, nbuf=2)]
  scatters = [_make_scatter(_EH, 0), _make_scatter(_EH, _EH)]
  zin = jnp.zeros((_RPT, HID), f32)

  xr = x.reshape(1, N_EDGES)
  ea = pl.pallas_call(
      _ea_body,
      grid=(_EGRIDX,),
      in_specs=[
          pl.BlockSpec((1, _EBX), lambda i: (0, i)),
          pl.BlockSpec((_EBX, HID), lambda i: (i, 0)),
          _vec_spec(HID), _vec_spec(HID),
          _mat_spec(HID, HID),
          _vec_spec(HID), _vec_spec(HID),
      ],
      out_specs=pl.BlockSpec((_EBX, HID), lambda i: (i, 0)),
      out_shape=jax.ShapeDtypeStruct((N_EDGES, HID), f32),
  )(xr, edge_attr, a1, d1, w2, g2, d2)

  def _msg_half(hs, half):
    off = half * _EGRID_H
    return pl.pallas_call(
        _msg_body,
        grid=(_EGRID_H,),
        in_specs=[pl.BlockSpec((_EB, HID), lambda i: (i, 0)),
                  pl.BlockSpec((_EB, HID), lambda i, o=off: (i + o, 0))],
        out_specs=pl.BlockSpec((_EB, HID), lambda i: (i, 0)),
        out_shape=jax.ShapeDtypeStruct((_EH, HID), f32),
    )(hs, ea)

  h = node_attr
  for li, name in enumerate(("conv1", "conv2", "conv3")):
    pc = params[name]
    gk, ck = _gate_bias(pc, t)
    dk = pc["b"][None] * gk + ck
    (hs0,) = gathers1[0](h, src)
    (hs1,) = gathers1[1](h, src)
    msg0 = _msg_half(hs0, 0)
    msg1 = _msg_half(hs1, 1)
    parts0 = scatters[0](msg0, dst, zin)
    parts1 = scatters[1](msg1, dst, zin)
    h = pl.pallas_call(
        functools.partial(_node_body, li < 2),
        grid=(_NGRID,),
        in_specs=[
            pl.BlockSpec((_NB, HID), lambda i: (i, 0)),
            pl.BlockSpec((_NB, HID), lambda i: (i + _NGRID, 0)),
            pl.BlockSpec((_NB, HID), lambda i: (i, 0)),
            pl.BlockSpec((_NB, HID), lambda i: (i + _NGRID, 0)),
            pl.BlockSpec((_NB, HID), lambda i: (i, 0)),
            _mat_spec(HID, HID),
            _vec_spec(HID), _vec_spec(HID),
        ],
        out_specs=pl.BlockSpec((_NB, HID), lambda i: (i, 0)),
        out_shape=jax.ShapeDtypeStruct((N_NODES, HID), f32),
    )(parts0, parts0, parts1, parts1, h, pc["W"], gk, dk)


  pf1, pf2, pf3 = params["out_fc1"], params["out_fc2"], params["out_fc3"]
  gf1, cf1 = _gate_bias(pf1, t)
  gf2, cf2 = _gate_bias(pf2, t)
  gf3, cf3 = _gate_bias(pf3, t)
  e1 = pf1["b"][None] * gf1 + cf1
  e2 = pf2["b"][None] * gf2 + cf2
  e3 = pf3["b"][None] * gf3 + cf3

  outs = []
  for half in range(2):
    hr, hc = gathers2[half](h, src, dst)
    off = half * _EGRID_H
    outs.append(pl.pallas_call(
        _pair_body,
        grid=(_EGRID_H,),
        in_specs=[
            pl.BlockSpec((_EB, HID), lambda i: (i, 0)),
            pl.BlockSpec((_EB, HID), lambda i: (i, 0)),
            pl.BlockSpec((_EB, HID), lambda i, o=off: (i + o, 0)),
            _mat_spec(HID, HID), _mat_spec(HID, HID),
            _vec_spec(HID), _vec_spec(HID),
            _mat_spec(HID, HID // 2),
            _vec_spec(HID // 2), _vec_spec(HID // 2),
            _mat_spec(HID // 2, 1),
            _vec_spec(1), _vec_spec(1),
        ],
        out_specs=pl.BlockSpec((_EB, 1), lambda i: (i, 0)),
        out_shape=jax.ShapeDtypeStruct((_EH, 1), f32),
    )(hr, hc, ea, pf1["W"][:HID], pf1["W"][HID:], gf1, e1,
      pf2["W"], gf2, e2, pf3["W"], gf3, e3))
  return jnp.concatenate(outs, axis=0)
